# baseline (speedup 1.0000x reference)
import functools

import jax
import jax.numpy as jnp
from jax import lax
from jax.experimental import pallas as pl
from jax.experimental.pallas import tpu as pltpu
from jax.experimental.pallas import tpu_sc as plsc

N = 10000
D = 128
E = 320000

NC = 2
NS = 16
NW = NC * NS

NP = 10240
HR = NP // 128
ROWS_PER_TILE = HR // NS

K_EDGE = 128
CH_K1 = 79
CH0 = 0
CH1 = 158
QC = 40
NQ = 4
FLAT_CH = 2536
E_PAD = FLAT_CH * K_EDGE
ROWS_OUT = NP // NS

_mesh = lambda: plsc.VectorSubcoreMesh(core_axis_name="c", subcore_axis_name="s")


def _newton_rsqrt(d):
    i = plsc.bitcast(d, jnp.int32)
    i = jnp.int32(0x5F3759DF) - lax.shift_right_logical(i, 1)
    y = plsc.bitcast(i, jnp.float32)
    for _ in range(3):
        y = y * (1.5 - 0.5 * d * y * y)
    return y


def _k1_body(dst_hbm, ones_hbm, deg_hbm, didx_v, ones_v, sem, deg_sh):
    c = lax.axis_index("c")
    s = lax.axis_index("s")
    w = c * NS + s
    npt = NP // NS
    zeros16 = jnp.zeros((16,), jnp.float32)

    def zbody(i, _):
        r = i // 8
        k = i % 8
        ones_v[r, pl.ds(k * 16, 16)] = zeros16
        return 0

    lax.fori_loop(0, K_EDGE * 8, zbody, 0)
    for t in range(npt // K_EDGE):
        pltpu.sync_copy(ones_v,
                        deg_sh.at[pl.ds(s * npt + t * K_EDGE, K_EDGE)])
    pltpu.sync_copy(ones_hbm, ones_v)
    pltpu.sync_copy(dst_hbm.at[w], didx_v)
    plsc.subcore_barrier()

    def fire(j, _):
        pltpu.async_copy(ones_v, deg_sh.at[didx_v.at[j]], sem, add=True)
        pltpu.make_async_copy(ones_v, deg_sh.at[didx_v.at[j]], sem).wait()
        return 0

    lax.fori_loop(0, CH_K1, fire, 0)
    plsc.subcore_barrier()

    pltpu.sync_copy(deg_sh.at[pl.ds(s * npt, npt)],
                    deg_hbm.at[c, pl.ds(s * npt, npt)])


def _run_k1(dst_r, ones_col):
    f = pl.kernel(
        _k1_body,
        out_type=jax.ShapeDtypeStruct((NC, NP, 128), jnp.float32),
        mesh=_mesh(),
        scratch_types=[
            pltpu.VMEM((CH_K1, K_EDGE), jnp.int32),
            pltpu.VMEM((K_EDGE, 128), jnp.float32),
            pltpu.SemaphoreType.DMA,
            pltpu.VMEM_SHARED((NP, 128), jnp.float32),
        ],
    )
    return f(dst_r, ones_col)


def _k2_body(x_ref, w0_ref, wih_ref, whh_ref, bih_ref, bhh_ref, deg_ref,
             hp_ref, w_s):
    @pl.when(pl.program_id(0) == 0)
    def _():
        w0 = w0_ref[...]
        wc = wih_ref[...] + whh_ref[...]
        b = bih_ref[...] + bhh_ref[...]
        gates = lax.dot_general(
            w0, wc, (((1,), (1,)), ((), ())),
            preferred_element_type=jnp.float32) + b
        i_ = jax.nn.sigmoid(gates[:, 0:128])
        f_ = jax.nn.sigmoid(gates[:, 128:256])
        g_ = jnp.tanh(gates[:, 256:384])
        o_ = jax.nn.sigmoid(gates[:, 384:512])
        c_ = f_ * w0 + i_ * g_
        w_s[...] = o_ * jnp.tanh(c_)

    h = jnp.dot(x_ref[...], w_s[...], preferred_element_type=jnp.float32)
    deg = deg_ref[0] + deg_ref[1] + 1.0
    hp_ref[...] = h * lax.rsqrt(deg)


def _run_k2(x, w0, wih, whh, bih, bhh, deg):
    rb = 400
    grid = (N // rb,)
    return pl.pallas_call(
        _k2_body,
        grid=grid,
        in_specs=[
            pl.BlockSpec((rb, D), lambda i: (i, 0)),
            pl.BlockSpec((D, D), lambda i: (0, 0)),
            pl.BlockSpec((4 * D, D), lambda i: (0, 0)),
            pl.BlockSpec((4 * D, D), lambda i: (0, 0)),
            pl.BlockSpec((1, 4 * D), lambda i: (0, 0)),
            pl.BlockSpec((1, 4 * D), lambda i: (0, 0)),
            pl.BlockSpec((NC, rb, 1), lambda i: (0, i, 0)),
        ],
        out_specs=pl.BlockSpec((rb, D), lambda i: (i, 0)),
        out_shape=jax.ShapeDtypeStruct((N, D), jnp.float32),
        scratch_shapes=[pltpu.VMEM((D, D), jnp.float32)],
    )(x, w0, wih, whh, bih, bhh, deg)


def _k3_body(src_hbm, dst_hbm, hp_hbm, acc_hbm,
             sidx_v, didx_v, rows0_v, rows1_v, gs0, gs1, ss0, ss1, acc_sh):
    c = lax.axis_index("c")
    s = lax.axis_index("s")
    w = c * NS + s
    zeros16 = jnp.zeros((16,), jnp.float32)

    def zbody(i, _):
        r = i // 8
        k = i % 8
        rows0_v[r, pl.ds(k * 16, 16)] = zeros16
        return 0

    lax.fori_loop(0, K_EDGE * 8, zbody, 0)
    for t in range(ROWS_OUT // K_EDGE):
        pltpu.sync_copy(
            rows0_v, acc_sh.at[pl.ds(s * ROWS_OUT + t * K_EDGE, K_EDGE)]
        )

    plsc.subcore_barrier()

    rows = (rows0_v, rows1_v)
    gsem = (gs0, gs1)
    ssem = (ss0, ss1)

    ch_c = jnp.where(c == 0, CH0, CH1)
    base_c = jnp.where(c == 0, s * CH0, NS * CH0 + s * CH1)

    for q in range(NQ):
        start = base_c + q * QC
        cnt = jnp.clip(ch_c - q * QC, 0, QC)

        @pl.when(cnt > 0)
        def _():
            aligned = (start // 8) * 8
            off = start - aligned
            pltpu.sync_copy(src_hbm.at[pl.ds(aligned, QC + 8)], sidx_v)
            pltpu.sync_copy(dst_hbm.at[pl.ds(aligned, QC + 8)], didx_v)
            for b in range(2):
                pltpu.async_copy(
                    hp_hbm.at[sidx_v.at[off + b]], rows[b], gsem[b])

            def pair(p, _):
                jl = p * 2
                for b in range(2):
                    j = jl + b
                    pltpu.make_async_copy(
                        hp_hbm.at[sidx_v.at[off + j]], rows[b],
                        gsem[b]).wait()
                    pltpu.async_copy(
                        rows[b], acc_sh.at[didx_v.at[off + j]], ssem[b],
                        add=True)
                for b in range(2):
                    j = jl + b
                    pltpu.make_async_copy(
                        rows[b], acc_sh.at[didx_v.at[off + j]],
                        ssem[b]).wait()

                    @pl.when(j + 2 < cnt)
                    def _():
                        pltpu.async_copy(
                            hp_hbm.at[sidx_v.at[off + j + 2]], rows[b],
                            gsem[b])

                return 0

            lax.fori_loop(0, cnt // 2, pair, 0)

    plsc.subcore_barrier()

    pltpu.sync_copy(
        acc_sh.at[pl.ds(s * ROWS_OUT, ROWS_OUT)],
        acc_hbm.at[c, pl.ds(s * ROWS_OUT, ROWS_OUT)],
    )


def _run_k3(src_f, dst_f, hp):
    f = pl.kernel(
        _k3_body,
        out_type=jax.ShapeDtypeStruct((NC, NP, D), jnp.float32),
        mesh=_mesh(),
        scratch_types=[
            pltpu.VMEM((QC + 8, K_EDGE), jnp.int32),
            pltpu.VMEM((QC + 8, K_EDGE), jnp.int32),
            pltpu.VMEM((K_EDGE, D), jnp.float32),
            pltpu.VMEM((K_EDGE, D), jnp.float32),
            pltpu.SemaphoreType.DMA,
            pltpu.SemaphoreType.DMA,
            pltpu.SemaphoreType.DMA,
            pltpu.SemaphoreType.DMA,
            pltpu.VMEM_SHARED((NP, D), jnp.float32),
        ],
    )
    return f(src_f, dst_f, hp)


def _k4_body(acc_ref, hp_ref, deg_ref, out_ref):
    tot = acc_ref[0] + acc_ref[1] + hp_ref[...]
    deg = deg_ref[0] + deg_ref[1] + 1.0
    out_ref[...] = jnp.maximum(tot * lax.rsqrt(deg), 0.0)


def _run_k4(acc, hp, deg):
    rb = 400
    return pl.pallas_call(
        _k4_body,
        grid=(N // rb,),
        in_specs=[
            pl.BlockSpec((NC, rb, D), lambda i: (0, i, 0)),
            pl.BlockSpec((rb, D), lambda i: (i, 0)),
            pl.BlockSpec((NC, rb, 1), lambda i: (0, i, 0)),
        ],
        out_specs=pl.BlockSpec((rb, D), lambda i: (i, 0)),
        out_shape=jax.ShapeDtypeStruct((N, D), jnp.float32),
    )(acc, hp, deg)


def kernel(x, edge_index, W0, W_ih, W_hh, b_ih, b_hh):
    pad = E_PAD - E
    src_f = jnp.concatenate(
        [edge_index[0], jnp.zeros((pad,), jnp.int32)]).reshape(FLAT_CH,
                                                               K_EDGE)
    dst_f = jnp.concatenate(
        [edge_index[1], jnp.full((pad,), N, jnp.int32)]).reshape(FLAT_CH,
                                                                 K_EDGE)
    dst_r = dst_f[:NW * CH_K1].reshape(NW, CH_K1, K_EDGE)

    ones_col = jnp.zeros((K_EDGE, 128), jnp.float32).at[:, 0].set(1.0)
    deg128 = _run_k1(dst_r, ones_col)
    deg = deg128[:, :, 0:1]

    hp = _run_k2(x, W0, W_ih, W_hh,
                 b_ih.reshape(1, 4 * D), b_hh.reshape(1, 4 * D), deg)

    acc = _run_k3(src_f, dst_f, hp)

    return _run_k4(acc, hp, deg)


# speedup vs baseline: 1.0226x; 1.0226x over previous
import functools

import jax
import jax.numpy as jnp
from jax import lax
from jax.experimental import pallas as pl
from jax.experimental.pallas import tpu as pltpu
from jax.experimental.pallas import tpu_sc as plsc

N = 10000
D = 128
E = 320000

NC = 2
NS = 16
NW = NC * NS

NP = 10240
HR = NP // 128
ROWS_PER_TILE = HR // NS

K_EDGE = 128
CH_K1 = 79
CH0 = 158
CH1 = 0
QC = 40
NQ = 4
FLAT_CH = 2536
E_PAD = FLAT_CH * K_EDGE
ROWS_OUT = NP // NS

_mesh = lambda: plsc.VectorSubcoreMesh(core_axis_name="c", subcore_axis_name="s")


def _newton_rsqrt(d):
    i = plsc.bitcast(d, jnp.int32)
    i = jnp.int32(0x5F3759DF) - lax.shift_right_logical(i, 1)
    y = plsc.bitcast(i, jnp.float32)
    for _ in range(3):
        y = y * (1.5 - 0.5 * d * y * y)
    return y


def _k1_body(dst_hbm, ones_hbm, deg_hbm, didx_v, ones_v, sem, deg_sh):
    c = lax.axis_index("c")
    s = lax.axis_index("s")
    w = c * NS + s
    npt = NP // NS
    zeros16 = jnp.zeros((16,), jnp.float32)

    def zbody(i, _):
        r = i // 8
        k = i % 8
        ones_v[r, pl.ds(k * 16, 16)] = zeros16
        return 0

    lax.fori_loop(0, K_EDGE * 8, zbody, 0)
    for t in range(npt // K_EDGE):
        pltpu.sync_copy(ones_v,
                        deg_sh.at[pl.ds(s * npt + t * K_EDGE, K_EDGE)])
    pltpu.sync_copy(ones_hbm, ones_v)
    pltpu.sync_copy(dst_hbm.at[w], didx_v)
    plsc.subcore_barrier()

    def fire(j, _):
        pltpu.async_copy(ones_v, deg_sh.at[didx_v.at[j]], sem, add=True)
        pltpu.make_async_copy(ones_v, deg_sh.at[didx_v.at[j]], sem).wait()
        return 0

    lax.fori_loop(0, CH_K1, fire, 0)
    plsc.subcore_barrier()

    pltpu.sync_copy(deg_sh.at[pl.ds(s * npt, npt)],
                    deg_hbm.at[c, pl.ds(s * npt, npt)])


def _run_k1(dst_r, ones_col):
    f = pl.kernel(
        _k1_body,
        out_type=jax.ShapeDtypeStruct((NC, NP, 128), jnp.float32),
        mesh=_mesh(),
        scratch_types=[
            pltpu.VMEM((CH_K1, K_EDGE), jnp.int32),
            pltpu.VMEM((K_EDGE, 128), jnp.float32),
            pltpu.SemaphoreType.DMA,
            pltpu.VMEM_SHARED((NP, 128), jnp.float32),
        ],
    )
    return f(dst_r, ones_col)


def _k2_body(x_ref, w0_ref, wih_ref, whh_ref, bih_ref, bhh_ref, deg_ref,
             hp_ref, w_s):
    @pl.when(pl.program_id(0) == 0)
    def _():
        w0 = w0_ref[...]
        wc = wih_ref[...] + whh_ref[...]
        b = bih_ref[...] + bhh_ref[...]
        gates = lax.dot_general(
            w0, wc, (((1,), (1,)), ((), ())),
            preferred_element_type=jnp.float32) + b
        i_ = jax.nn.sigmoid(gates[:, 0:128])
        f_ = jax.nn.sigmoid(gates[:, 128:256])
        g_ = jnp.tanh(gates[:, 256:384])
        o_ = jax.nn.sigmoid(gates[:, 384:512])
        c_ = f_ * w0 + i_ * g_
        w_s[...] = o_ * jnp.tanh(c_)

    h = jnp.dot(x_ref[...], w_s[...], preferred_element_type=jnp.float32)
    deg = deg_ref[0] + deg_ref[1] + 1.0
    hp_ref[...] = h * lax.rsqrt(deg)


def _run_k2(x, w0, wih, whh, bih, bhh, deg):
    rb = 400
    grid = (N // rb,)
    return pl.pallas_call(
        _k2_body,
        grid=grid,
        in_specs=[
            pl.BlockSpec((rb, D), lambda i: (i, 0)),
            pl.BlockSpec((D, D), lambda i: (0, 0)),
            pl.BlockSpec((4 * D, D), lambda i: (0, 0)),
            pl.BlockSpec((4 * D, D), lambda i: (0, 0)),
            pl.BlockSpec((1, 4 * D), lambda i: (0, 0)),
            pl.BlockSpec((1, 4 * D), lambda i: (0, 0)),
            pl.BlockSpec((NC, rb, 1), lambda i: (0, i, 0)),
        ],
        out_specs=pl.BlockSpec((rb, D), lambda i: (i, 0)),
        out_shape=jax.ShapeDtypeStruct((N, D), jnp.float32),
        scratch_shapes=[pltpu.VMEM((D, D), jnp.float32)],
    )(x, w0, wih, whh, bih, bhh, deg)


def _k3_body(src_hbm, dst_hbm, hp_hbm, acc_hbm,
             sidx_v, didx_v, rows0_v, rows1_v, gs0, gs1, ss0, ss1, acc_sh):
    c = lax.axis_index("c")
    s = lax.axis_index("s")
    w = c * NS + s
    zeros16 = jnp.zeros((16,), jnp.float32)

    def zbody(i, _):
        r = i // 8
        k = i % 8
        rows0_v[r, pl.ds(k * 16, 16)] = zeros16
        return 0

    lax.fori_loop(0, K_EDGE * 8, zbody, 0)
    for t in range(ROWS_OUT // K_EDGE):
        pltpu.sync_copy(
            rows0_v, acc_sh.at[pl.ds(s * ROWS_OUT + t * K_EDGE, K_EDGE)]
        )

    plsc.subcore_barrier()

    rows = (rows0_v, rows1_v)
    gsem = (gs0, gs1)
    ssem = (ss0, ss1)

    ch_c = jnp.where(c == 0, CH0, CH1)
    base_c = jnp.where(c == 0, s * CH0, NS * CH0 + s * CH1)

    for q in range(NQ):
        start = base_c + q * QC
        cnt = jnp.clip(ch_c - q * QC, 0, QC)

        @pl.when(cnt > 0)
        def _():
            aligned = (start // 8) * 8
            off = start - aligned
            pltpu.sync_copy(src_hbm.at[pl.ds(aligned, QC + 8)], sidx_v)
            pltpu.sync_copy(dst_hbm.at[pl.ds(aligned, QC + 8)], didx_v)
            for b in range(2):
                pltpu.async_copy(
                    hp_hbm.at[sidx_v.at[off + b]], rows[b], gsem[b])

            def pair(p, _):
                jl = p * 2
                for b in range(2):
                    j = jl + b
                    pltpu.make_async_copy(
                        hp_hbm.at[sidx_v.at[off + j]], rows[b],
                        gsem[b]).wait()
                    pltpu.async_copy(
                        rows[b], acc_sh.at[didx_v.at[off + j]], ssem[b],
                        add=True)
                for b in range(2):
                    j = jl + b
                    pltpu.make_async_copy(
                        rows[b], acc_sh.at[didx_v.at[off + j]],
                        ssem[b]).wait()

                    @pl.when(j + 2 < cnt)
                    def _():
                        pltpu.async_copy(
                            hp_hbm.at[sidx_v.at[off + j + 2]], rows[b],
                            gsem[b])

                return 0

            lax.fori_loop(0, cnt // 2, pair, 0)

    plsc.subcore_barrier()

    pltpu.sync_copy(
        acc_sh.at[pl.ds(s * ROWS_OUT, ROWS_OUT)],
        acc_hbm.at[c, pl.ds(s * ROWS_OUT, ROWS_OUT)],
    )


def _run_k3(src_f, dst_f, hp):
    f = pl.kernel(
        _k3_body,
        out_type=jax.ShapeDtypeStruct((NC, NP, D), jnp.float32),
        mesh=_mesh(),
        scratch_types=[
            pltpu.VMEM((QC + 8, K_EDGE), jnp.int32),
            pltpu.VMEM((QC + 8, K_EDGE), jnp.int32),
            pltpu.VMEM((K_EDGE, D), jnp.float32),
            pltpu.VMEM((K_EDGE, D), jnp.float32),
            pltpu.SemaphoreType.DMA,
            pltpu.SemaphoreType.DMA,
            pltpu.SemaphoreType.DMA,
            pltpu.SemaphoreType.DMA,
            pltpu.VMEM_SHARED((NP, D), jnp.float32),
        ],
    )
    return f(src_f, dst_f, hp)


def _k4_body(acc_ref, hp_ref, deg_ref, out_ref):
    tot = acc_ref[0] + acc_ref[1] + hp_ref[...]
    deg = deg_ref[0] + deg_ref[1] + 1.0
    out_ref[...] = jnp.maximum(tot * lax.rsqrt(deg), 0.0)


def _run_k4(acc, hp, deg):
    rb = 400
    return pl.pallas_call(
        _k4_body,
        grid=(N // rb,),
        in_specs=[
            pl.BlockSpec((NC, rb, D), lambda i: (0, i, 0)),
            pl.BlockSpec((rb, D), lambda i: (i, 0)),
            pl.BlockSpec((NC, rb, 1), lambda i: (0, i, 0)),
        ],
        out_specs=pl.BlockSpec((rb, D), lambda i: (i, 0)),
        out_shape=jax.ShapeDtypeStruct((N, D), jnp.float32),
    )(acc, hp, deg)


def kernel(x, edge_index, W0, W_ih, W_hh, b_ih, b_hh):
    pad = E_PAD - E
    src_f = jnp.concatenate(
        [edge_index[0], jnp.zeros((pad,), jnp.int32)]).reshape(FLAT_CH,
                                                               K_EDGE)
    dst_f = jnp.concatenate(
        [edge_index[1], jnp.full((pad,), N, jnp.int32)]).reshape(FLAT_CH,
                                                                 K_EDGE)
    dst_r = dst_f[:NW * CH_K1].reshape(NW, CH_K1, K_EDGE)

    ones_col = jnp.zeros((K_EDGE, 128), jnp.float32).at[:, 0].set(1.0)
    deg128 = _run_k1(dst_r, ones_col)
    deg = deg128[:, :, 0:1]

    hp = _run_k2(x, W0, W_ih, W_hh,
                 b_ih.reshape(1, 4 * D), b_hh.reshape(1, 4 * D), deg)

    acc = _run_k3(src_f, dst_f, hp)

    return _run_k4(acc, hp, deg)


# speedup vs baseline: 1.1609x; 1.1352x over previous
import functools

import jax
import jax.numpy as jnp
from jax import lax
from jax.experimental import pallas as pl
from jax.experimental.pallas import tpu as pltpu
from jax.experimental.pallas import tpu_sc as plsc

N = 10000
D = 128
E = 320000

NC = 2
NS = 16
NW = NC * NS

NP = 10240
HR = NP // 128
ROWS_PER_TILE = HR // NS

K_EDGE = 128
CH_K1 = 79
CH0 = 79
CH1 = 79
QC = 40
NQ = 2
FLAT_CH = 2536
E_PAD = FLAT_CH * K_EDGE
ROWS_OUT = NP // NS

_mesh = lambda: plsc.VectorSubcoreMesh(core_axis_name="c", subcore_axis_name="s")


def _newton_rsqrt(d):
    i = plsc.bitcast(d, jnp.int32)
    i = jnp.int32(0x5F3759DF) - lax.shift_right_logical(i, 1)
    y = plsc.bitcast(i, jnp.float32)
    for _ in range(3):
        y = y * (1.5 - 0.5 * d * y * y)
    return y


def _k1_body(dst_hbm, ones_hbm, deg_hbm, didx_v, ones_v, sem, deg_sh):
    c = lax.axis_index("c")
    s = lax.axis_index("s")
    w = c * NS + s
    npt = NP // NS
    zeros16 = jnp.zeros((16,), jnp.float32)

    def zbody(i, _):
        r = i // 8
        k = i % 8
        ones_v[r, pl.ds(k * 16, 16)] = zeros16
        return 0

    lax.fori_loop(0, K_EDGE * 8, zbody, 0)
    for t in range(npt // K_EDGE):
        pltpu.sync_copy(ones_v,
                        deg_sh.at[pl.ds(s * npt + t * K_EDGE, K_EDGE)])
    pltpu.sync_copy(ones_hbm, ones_v)
    pltpu.sync_copy(dst_hbm.at[w], didx_v)
    plsc.subcore_barrier()

    def fire4(p, _):
        j0 = p * 4
        for b in range(4):
            pltpu.async_copy(ones_v, deg_sh.at[didx_v.at[j0 + b]], sem,
                             add=True)
        for b in range(4):
            pltpu.make_async_copy(ones_v, deg_sh.at[didx_v.at[j0 + b]],
                                  sem).wait()
        return 0

    lax.fori_loop(0, CH_K1 // 4, fire4, 0)
    for j in range(CH_K1 - (CH_K1 // 4) * 4):
        jj = (CH_K1 // 4) * 4 + j
        pltpu.async_copy(ones_v, deg_sh.at[didx_v.at[jj]], sem, add=True)
        pltpu.make_async_copy(ones_v, deg_sh.at[didx_v.at[jj]], sem).wait()
    plsc.subcore_barrier()

    pltpu.sync_copy(deg_sh.at[pl.ds(s * npt, npt)],
                    deg_hbm.at[c, pl.ds(s * npt, npt)])


def _run_k1(dst_r, ones_col):
    f = pl.kernel(
        _k1_body,
        out_type=jax.ShapeDtypeStruct((NC, NP, 128), jnp.float32),
        mesh=_mesh(),
        scratch_types=[
            pltpu.VMEM((CH_K1, K_EDGE), jnp.int32),
            pltpu.VMEM((K_EDGE, 128), jnp.float32),
            pltpu.SemaphoreType.DMA,
            pltpu.VMEM_SHARED((NP, 128), jnp.float32),
        ],
    )
    return f(dst_r, ones_col)


def _k2_body(x_ref, w0_ref, wih_ref, whh_ref, bih_ref, bhh_ref, deg_ref,
             hp_ref, w_s):
    @pl.when(pl.program_id(0) == 0)
    def _():
        w0 = w0_ref[...]
        wc = wih_ref[...] + whh_ref[...]
        b = bih_ref[...] + bhh_ref[...]
        gates = lax.dot_general(
            w0, wc, (((1,), (1,)), ((), ())),
            preferred_element_type=jnp.float32) + b
        i_ = jax.nn.sigmoid(gates[:, 0:128])
        f_ = jax.nn.sigmoid(gates[:, 128:256])
        g_ = jnp.tanh(gates[:, 256:384])
        o_ = jax.nn.sigmoid(gates[:, 384:512])
        c_ = f_ * w0 + i_ * g_
        w_s[...] = o_ * jnp.tanh(c_)

    h = jnp.dot(x_ref[...], w_s[...], preferred_element_type=jnp.float32)
    deg = deg_ref[0] + deg_ref[1] + 1.0
    hp_ref[...] = h * lax.rsqrt(deg)


def _run_k2(x, w0, wih, whh, bih, bhh, deg):
    rb = 400
    grid = (N // rb,)
    return pl.pallas_call(
        _k2_body,
        grid=grid,
        in_specs=[
            pl.BlockSpec((rb, D), lambda i: (i, 0)),
            pl.BlockSpec((D, D), lambda i: (0, 0)),
            pl.BlockSpec((4 * D, D), lambda i: (0, 0)),
            pl.BlockSpec((4 * D, D), lambda i: (0, 0)),
            pl.BlockSpec((1, 4 * D), lambda i: (0, 0)),
            pl.BlockSpec((1, 4 * D), lambda i: (0, 0)),
            pl.BlockSpec((NC, rb, 1), lambda i: (0, i, 0)),
        ],
        out_specs=pl.BlockSpec((rb, D), lambda i: (i, 0)),
        out_shape=jax.ShapeDtypeStruct((N, D), jnp.float32),
        scratch_shapes=[pltpu.VMEM((D, D), jnp.float32)],
    )(x, w0, wih, whh, bih, bhh, deg)


def _k3_body(src_hbm, dst_hbm, hp_hbm, acc_hbm,
             sidx_v, didx_v, rows0_v, gs0, ss0, acc_sh):
    c = lax.axis_index("c")
    s = lax.axis_index("s")
    w = c * NS + s
    zeros16 = jnp.zeros((16,), jnp.float32)

    def zbody(i, _):
        r = i // 8
        k = i % 8
        rows0_v[r, pl.ds(k * 16, 16)] = zeros16
        return 0

    lax.fori_loop(0, K_EDGE * 8, zbody, 0)
    for t in range(ROWS_OUT // K_EDGE):
        pltpu.sync_copy(
            rows0_v, acc_sh.at[pl.ds(s * ROWS_OUT + t * K_EDGE, K_EDGE)]
        )

    plsc.subcore_barrier()

    ch_c = jnp.where(c == 0, CH0, CH1)
    base_c = jnp.where(c == 0, s * CH0, NS * CH0 + s * CH1)

    for q in range(NQ):
        start = base_c + q * QC
        cnt = jnp.clip(ch_c - q * QC, 0, QC)

        @pl.when(cnt > 0)
        def _():
            aligned = (start // 8) * 8
            off = start - aligned
            pltpu.sync_copy(src_hbm.at[pl.ds(aligned, QC + 8)], sidx_v)
            pltpu.sync_copy(dst_hbm.at[pl.ds(aligned, QC + 8)], didx_v)

            def chunk(jj, _):
                j = off + jj
                pltpu.async_copy(hp_hbm.at[sidx_v.at[j]], rows0_v, gs0)
                pltpu.make_async_copy(
                    hp_hbm.at[sidx_v.at[j]], rows0_v, gs0).wait()
                pltpu.async_copy(
                    rows0_v, acc_sh.at[didx_v.at[j]], ss0, add=True)
                pltpu.make_async_copy(
                    rows0_v, acc_sh.at[didx_v.at[j]], ss0).wait()
                return 0

            lax.fori_loop(0, cnt, chunk, 0)

    plsc.subcore_barrier()

    pltpu.sync_copy(
        acc_sh.at[pl.ds(s * ROWS_OUT, ROWS_OUT)],
        acc_hbm.at[c, pl.ds(s * ROWS_OUT, ROWS_OUT)],
    )


def _run_k3(src_f, dst_f, hp):
    f = pl.kernel(
        _k3_body,
        out_type=jax.ShapeDtypeStruct((NC, NP, D), jnp.float32),
        mesh=_mesh(),
        scratch_types=[
            pltpu.VMEM((QC + 8, K_EDGE), jnp.int32),
            pltpu.VMEM((QC + 8, K_EDGE), jnp.int32),
            pltpu.VMEM((K_EDGE, D), jnp.float32),
            pltpu.SemaphoreType.DMA,
            pltpu.SemaphoreType.DMA,
            pltpu.VMEM_SHARED((NP, D), jnp.float32),
        ],
    )
    return f(src_f, dst_f, hp)


def _k4_body(acc_ref, hp_ref, deg_ref, out_ref):
    tot = acc_ref[0] + acc_ref[1] + hp_ref[...]
    deg = deg_ref[0] + deg_ref[1] + 1.0
    out_ref[...] = jnp.maximum(tot * lax.rsqrt(deg), 0.0)


def _run_k4(acc, hp, deg):
    rb = 400
    return pl.pallas_call(
        _k4_body,
        grid=(N // rb,),
        in_specs=[
            pl.BlockSpec((NC, rb, D), lambda i: (0, i, 0)),
            pl.BlockSpec((rb, D), lambda i: (i, 0)),
            pl.BlockSpec((NC, rb, 1), lambda i: (0, i, 0)),
        ],
        out_specs=pl.BlockSpec((rb, D), lambda i: (i, 0)),
        out_shape=jax.ShapeDtypeStruct((N, D), jnp.float32),
    )(acc, hp, deg)


def kernel(x, edge_index, W0, W_ih, W_hh, b_ih, b_hh):
    pad = E_PAD - E
    src_f = jnp.concatenate(
        [edge_index[0], jnp.zeros((pad,), jnp.int32)]).reshape(FLAT_CH,
                                                               K_EDGE)
    dst_f = jnp.concatenate(
        [edge_index[1], jnp.full((pad,), N, jnp.int32)]).reshape(FLAT_CH,
                                                                 K_EDGE)
    dst_r = dst_f[:NW * CH_K1].reshape(NW, CH_K1, K_EDGE)

    ones_col = jnp.zeros((K_EDGE, 128), jnp.float32).at[:, 0].set(1.0)
    deg128 = _run_k1(dst_r, ones_col)
    deg = deg128[:, :, 0:1]

    hp = _run_k2(x, W0, W_ih, W_hh,
                 b_ih.reshape(1, 4 * D), b_hh.reshape(1, 4 * D), deg)

    acc = _run_k3(src_f, dst_f, hp)

    return _run_k4(acc, hp, deg)


# speedup vs baseline: 1.3150x; 1.1328x over previous
import functools

import jax
import jax.numpy as jnp
from jax import lax
from jax.experimental import pallas as pl
from jax.experimental.pallas import tpu as pltpu
from jax.experimental.pallas import tpu_sc as plsc

N = 10000
D = 128
E = 320000

NC = 2
NS = 16
NW = NC * NS

NP = 10240
HR = NP // 128
ROWS_PER_TILE = HR // NS

K_EDGE = 128
CH_K1 = 79
CH0 = 104
CH1 = 54
QC = 40
NQ = 3
FLAT_CH = 2568
E_PAD = FLAT_CH * K_EDGE
ROWS_OUT = NP // NS

_mesh = lambda: plsc.VectorSubcoreMesh(core_axis_name="c", subcore_axis_name="s")


def _newton_rsqrt(d):
    i = plsc.bitcast(d, jnp.int32)
    i = jnp.int32(0x5F3759DF) - lax.shift_right_logical(i, 1)
    y = plsc.bitcast(i, jnp.float32)
    for _ in range(3):
        y = y * (1.5 - 0.5 * d * y * y)
    return y


def _k1_body(dst_hbm, ones_hbm, deg_hbm, didx_v, ones_v, sem, deg_sh):
    c = lax.axis_index("c")
    s = lax.axis_index("s")
    w = c * NS + s
    npt = NP // NS
    zeros16 = jnp.zeros((16,), jnp.float32)

    def zbody(i, _):
        r = i // 8
        k = i % 8
        ones_v[r, pl.ds(k * 16, 16)] = zeros16
        return 0

    lax.fori_loop(0, K_EDGE * 8, zbody, 0)
    for t in range(npt // K_EDGE):
        pltpu.sync_copy(ones_v,
                        deg_sh.at[pl.ds(s * npt + t * K_EDGE, K_EDGE)])
    pltpu.sync_copy(ones_hbm, ones_v)
    pltpu.sync_copy(dst_hbm.at[w], didx_v)
    plsc.subcore_barrier()

    def fire4(p, _):
        j0 = p * 4
        for b in range(4):
            pltpu.async_copy(ones_v, deg_sh.at[didx_v.at[j0 + b]], sem,
                             add=True)
        for b in range(4):
            pltpu.make_async_copy(ones_v, deg_sh.at[didx_v.at[j0 + b]],
                                  sem).wait()
        return 0

    lax.fori_loop(0, CH_K1 // 4, fire4, 0)
    for j in range(CH_K1 - (CH_K1 // 4) * 4):
        jj = (CH_K1 // 4) * 4 + j
        pltpu.async_copy(ones_v, deg_sh.at[didx_v.at[jj]], sem, add=True)
        pltpu.make_async_copy(ones_v, deg_sh.at[didx_v.at[jj]], sem).wait()
    plsc.subcore_barrier()

    pltpu.sync_copy(deg_sh.at[pl.ds(s * npt, npt)],
                    deg_hbm.at[c, pl.ds(s * npt, npt)])


def _run_k1(dst_r, ones_col):
    f = pl.kernel(
        _k1_body,
        out_type=jax.ShapeDtypeStruct((NC, NP, 128), jnp.float32),
        mesh=_mesh(),
        scratch_types=[
            pltpu.VMEM((CH_K1, K_EDGE), jnp.int32),
            pltpu.VMEM((K_EDGE, 128), jnp.float32),
            pltpu.SemaphoreType.DMA,
            pltpu.VMEM_SHARED((NP, 128), jnp.float32),
        ],
    )
    return f(dst_r, ones_col)


def _k2_body(x_ref, w0_ref, wih_ref, whh_ref, bih_ref, bhh_ref, deg_ref,
             hp_ref, w_s):
    @pl.when(pl.program_id(0) == 0)
    def _():
        w0 = w0_ref[...]
        wc = wih_ref[...] + whh_ref[...]
        b = bih_ref[...] + bhh_ref[...]
        gates = lax.dot_general(
            w0, wc, (((1,), (1,)), ((), ())),
            preferred_element_type=jnp.float32) + b
        i_ = jax.nn.sigmoid(gates[:, 0:128])
        f_ = jax.nn.sigmoid(gates[:, 128:256])
        g_ = jnp.tanh(gates[:, 256:384])
        o_ = jax.nn.sigmoid(gates[:, 384:512])
        c_ = f_ * w0 + i_ * g_
        w_s[...] = o_ * jnp.tanh(c_)

    h = jnp.dot(x_ref[...], w_s[...], preferred_element_type=jnp.float32)
    deg = deg_ref[0] + deg_ref[1] + 1.0
    hp_ref[...] = h * lax.rsqrt(deg)


def _run_k2(x, w0, wih, whh, bih, bhh, deg):
    rb = 400
    grid = (N // rb,)
    return pl.pallas_call(
        _k2_body,
        grid=grid,
        in_specs=[
            pl.BlockSpec((rb, D), lambda i: (i, 0)),
            pl.BlockSpec((D, D), lambda i: (0, 0)),
            pl.BlockSpec((4 * D, D), lambda i: (0, 0)),
            pl.BlockSpec((4 * D, D), lambda i: (0, 0)),
            pl.BlockSpec((1, 4 * D), lambda i: (0, 0)),
            pl.BlockSpec((1, 4 * D), lambda i: (0, 0)),
            pl.BlockSpec((NC, rb, 1), lambda i: (0, i, 0)),
        ],
        out_specs=pl.BlockSpec((rb, D), lambda i: (i, 0)),
        out_shape=jax.ShapeDtypeStruct((N, D), jnp.float32),
        scratch_shapes=[pltpu.VMEM((D, D), jnp.float32)],
    )(x, w0, wih, whh, bih, bhh, deg)


def _k3_body(src_hbm, dst_hbm, hp_hbm, acc_hbm,
             sidx_v, didx_v, rows0_v, gs0, ss0, acc_sh):
    c = lax.axis_index("c")
    s = lax.axis_index("s")
    w = c * NS + s
    zeros16 = jnp.zeros((16,), jnp.float32)

    def zbody(i, _):
        r = i // 8
        k = i % 8
        rows0_v[r, pl.ds(k * 16, 16)] = zeros16
        return 0

    lax.fori_loop(0, K_EDGE * 8, zbody, 0)
    for t in range(ROWS_OUT // K_EDGE):
        pltpu.sync_copy(
            rows0_v, acc_sh.at[pl.ds(s * ROWS_OUT + t * K_EDGE, K_EDGE)]
        )

    plsc.subcore_barrier()

    ch_c = jnp.where(c == 0, CH0, CH1)
    base_c = jnp.where(c == 0, s * CH0, NS * CH0 + s * CH1)

    for q in range(NQ):
        start = base_c + q * QC
        cnt = jnp.clip(ch_c - q * QC, 0, QC)

        @pl.when(cnt > 0)
        def _():
            aligned = (start // 8) * 8
            off = start - aligned
            pltpu.sync_copy(src_hbm.at[pl.ds(aligned, QC + 8)], sidx_v)
            pltpu.sync_copy(dst_hbm.at[pl.ds(aligned, QC + 8)], didx_v)

            def chunk(jj, _):
                j = off + jj
                pltpu.async_copy(hp_hbm.at[sidx_v.at[j]], rows0_v, gs0)
                pltpu.make_async_copy(
                    hp_hbm.at[sidx_v.at[j]], rows0_v, gs0).wait()
                pltpu.async_copy(
                    rows0_v, acc_sh.at[didx_v.at[j]], ss0, add=True)
                pltpu.make_async_copy(
                    rows0_v, acc_sh.at[didx_v.at[j]], ss0).wait()
                return 0

            lax.fori_loop(0, cnt, chunk, 0)

    plsc.subcore_barrier()

    pltpu.sync_copy(
        acc_sh.at[pl.ds(s * ROWS_OUT, ROWS_OUT)],
        acc_hbm.at[c, pl.ds(s * ROWS_OUT, ROWS_OUT)],
    )


def _run_k3(src_f, dst_f, hp):
    f = pl.kernel(
        _k3_body,
        out_type=jax.ShapeDtypeStruct((NC, NP, D), jnp.float32),
        mesh=_mesh(),
        scratch_types=[
            pltpu.VMEM((QC + 8, K_EDGE), jnp.int32),
            pltpu.VMEM((QC + 8, K_EDGE), jnp.int32),
            pltpu.VMEM((K_EDGE, D), jnp.float32),
            pltpu.SemaphoreType.DMA,
            pltpu.SemaphoreType.DMA,
            pltpu.VMEM_SHARED((NP, D), jnp.float32),
        ],
    )
    return f(src_f, dst_f, hp)


def _k4_body(acc_ref, hp_ref, deg_ref, out_ref):
    tot = acc_ref[0] + acc_ref[1] + hp_ref[...]
    deg = deg_ref[0] + deg_ref[1] + 1.0
    out_ref[...] = jnp.maximum(tot * lax.rsqrt(deg), 0.0)


def _run_k4(acc, hp, deg):
    rb = 400
    return pl.pallas_call(
        _k4_body,
        grid=(N // rb,),
        in_specs=[
            pl.BlockSpec((NC, rb, D), lambda i: (0, i, 0)),
            pl.BlockSpec((rb, D), lambda i: (i, 0)),
            pl.BlockSpec((NC, rb, 1), lambda i: (0, i, 0)),
        ],
        out_specs=pl.BlockSpec((rb, D), lambda i: (i, 0)),
        out_shape=jax.ShapeDtypeStruct((N, D), jnp.float32),
    )(acc, hp, deg)


def kernel(x, edge_index, W0, W_ih, W_hh, b_ih, b_hh):
    pad = E_PAD - E
    src_f = jnp.concatenate(
        [edge_index[0], jnp.zeros((pad,), jnp.int32)]).reshape(FLAT_CH,
                                                               K_EDGE)
    dst_f = jnp.concatenate(
        [edge_index[1], jnp.full((pad,), N, jnp.int32)]).reshape(FLAT_CH,
                                                                 K_EDGE)
    dst_r = dst_f[:NW * CH_K1].reshape(NW, CH_K1, K_EDGE)

    ones_col = jnp.zeros((K_EDGE, 128), jnp.float32).at[:, 0].set(1.0)
    deg128 = _run_k1(dst_r, ones_col)
    deg = deg128[:, :, 0:1]

    hp = _run_k2(x, W0, W_ih, W_hh,
                 b_ih.reshape(1, 4 * D), b_hh.reshape(1, 4 * D), deg)

    acc = _run_k3(src_f, dst_f, hp)

    return _run_k4(acc, hp, deg)
